# baseline (speedup 1.0000x reference)
import functools

import jax
import jax.numpy as jnp
from jax import lax
from jax.experimental import pallas as pl
from jax.experimental.pallas import tpu as pltpu
from jax.experimental.pallas import tpu_sc as plsc

_K = 256
_R = 64
_N = 8192
_L = 16
_NV = _N // _L
_M31 = 0x7FFFFFFF


def _keys(b):
    return b ^ ((b >> 31) & _M31)


def _sc_body(x_hbm, o_hbm, krow, orow, hist, ck, cc):
    wid = lax.axis_index("s") * 2 + lax.axis_index("c")
    lane = lax.iota(jnp.int32, _L)
    lane256 = lane * 256
    ones16 = jnp.ones((_L,), jnp.int32)
    zeros16 = jnp.zeros((_L,), jnp.int32)

    def do_row(r, _):
        row = wid * 2 + r
        pltpu.sync_copy(x_hbm.at[row], krow)

        def zh(i, c):
            hist[pl.ds(i * _L, _L)] = zeros16
            return c

        lax.fori_loop(0, 256, zh, 0)

        def pa(v, c):
            k = _keys(krow[pl.ds(v * _L, _L)])
            krow[pl.ds(v * _L, _L)] = k
            b8 = (k >> 24) + 128
            plsc.addupdate_scatter(hist, [lane256 + b8], ones16)
            return c

        lax.fori_loop(0, _NV, pa, 0)

        def pb(i, carry):
            found, bb, cntgt, above = carry
            g = 15 - i
            t = hist[pl.ds(g * _L, _L)]
            for l in range(1, _L):
                t = t + hist[pl.ds(l * 256 + g * _L, _L)]
            rs = lax.rev(plsc.cumsum(lax.rev(t, (0,))), (0,))
            s = rs + above
            h = jnp.sum((s >= _K).astype(jnp.int32))
            hit = (found == 0) & (h > 0)
            bb = jnp.where(hit, g * _L + h - 1, bb)
            cnt_here = jnp.sum(jnp.where(lane >= h, t, 0))
            cntgt = jnp.where(hit, above + cnt_here, cntgt)
            found = jnp.where(hit, jnp.int32(1), found)
            above = above + jnp.sum(t)
            return found, bb, cntgt, above

        _, bb, cnt_gt, _ = lax.fori_loop(
            0, 16, pb,
            (jnp.int32(0), jnp.int32(0), jnp.int32(0), jnp.int32(0)))
        need = _K - cnt_gt

        def pe(v, off):
            k = krow[pl.ds(v * _L, _L)]
            b8 = (k >> 24) + 128
            orow[pl.ds(v * _L, _L)] = jnp.where(
                b8 > bb, jnp.maximum(k, 0), 0)
            meq = b8 == bb
            idx = off + plsc.cumsum(meq.astype(jnp.int32)) - 1
            plsc.store_scatter(ck, [idx], k, mask=meq)
            plsc.store_scatter(cc, [idx], v * _L + lane, mask=meq)
            return off + plsc.all_reduce_population_count(meq)

        ncand = lax.fori_loop(0, _NV, pe, zeros16)
        n_sc = jnp.max(ncand)
        nv_c = (n_sc + _L - 1) // _L

        def count_cmp(c, strict):
            def inner(i, acc):
                kv = ck[pl.ds(i * _L, _L)]
                valid = (i * _L + lane) < ncand
                m = ((kv > c) if strict else (kv >= c)) & valid
                return acc + m.astype(jnp.int32)

            return jnp.sum(lax.fori_loop(0, nv_c, inner, zeros16))

        def pd(j, p):
            c = p | (jnp.int32(1) << (23 - j))
            return jnp.where(count_cmp(c, False) >= need, c, p)

        tkey = lax.fori_loop(0, 24, pd, (bb - 128) << 24)
        n_eq = need - count_cmp(tkey, True)

        def pf(i, racc):
            kv = ck[pl.ds(i * _L, _L)]
            cv = cc[pl.ds(i * _L, _L)]
            valid = (i * _L + lane) < ncand
            meq = (kv == tkey) & valid
            inc_eq = meq & ((racc + plsc.cumsum(meq.astype(jnp.int32)))
                            <= n_eq)
            mwin = ((kv > tkey) & valid) | inc_eq
            plsc.store_scatter(orow, [cv], jnp.maximum(kv, 0), mask=mwin)
            return racc + plsc.all_reduce_population_count(meq)

        lax.fori_loop(0, nv_c, pf, zeros16)
        pltpu.sync_copy(orow, o_hbm.at[row])
        return _

    lax.fori_loop(0, 2, do_row, 0)


def _build(interpret=False):
    mesh = plsc.VectorSubcoreMesh(
        core_axis_name="c", subcore_axis_name="s",
        num_cores=2, num_subcores=16)
    return pl.kernel(
        _sc_body,
        out_type=jax.ShapeDtypeStruct((_R, _N), jnp.int32),
        mesh=mesh,
        scratch_types=[
            pltpu.VMEM((_N,), jnp.int32),
            pltpu.VMEM((_N,), jnp.int32),
            pltpu.VMEM((16 * 256,), jnp.int32),
            pltpu.VMEM((_N,), jnp.int32),
            pltpu.VMEM((_N,), jnp.int32),
        ],
        compiler_params=pltpu.CompilerParams(needs_layout_passes=False),
        interpret=interpret,
    )


@jax.jit
def kernel(x):
    xi = jax.lax.bitcast_convert_type(x, jnp.int32)
    oi = _build()(xi)
    return jax.lax.bitcast_convert_type(oi, jnp.float32)


# speedup vs baseline: 1.0169x; 1.0169x over previous
import functools

import jax
import jax.numpy as jnp
from jax import lax
from jax.experimental import pallas as pl
from jax.experimental.pallas import tpu as pltpu
from jax.experimental.pallas import tpu_sc as plsc

_K = 256
_R = 64
_N = 8192
_L = 16
_NV = _N // _L
_M31 = 0x7FFFFFFF


def _keys(b):
    return b ^ ((b >> 31) & _M31)


def _sc_body(x_hbm, o_hbm, krow, orow, hist, ck, cc):
    wid = lax.axis_index("s") * 2 + lax.axis_index("c")
    lane = lax.iota(jnp.int32, _L)
    lane256 = lane * 256
    ones16 = jnp.ones((_L,), jnp.int32)
    zeros16 = jnp.zeros((_L,), jnp.int32)

    def do_row(r, _):
        row = wid * 2 + r
        pltpu.sync_copy(x_hbm.at[row], krow)

        def zh(i, c):
            hist[pl.ds(i * _L, _L)] = zeros16
            return c

        lax.fori_loop(0, 256, zh, 0, unroll=8)

        def pa(v, c):
            k = _keys(krow[pl.ds(v * _L, _L)])
            krow[pl.ds(v * _L, _L)] = k
            b8 = (k >> 24) + 128
            plsc.addupdate_scatter(hist, [lane256 + b8], ones16)
            return c

        lax.fori_loop(0, _NV, pa, 0, unroll=8)

        def pb(i, carry):
            found, bb, cntgt, above = carry
            g = 15 - i
            t = hist[pl.ds(g * _L, _L)]
            for l in range(1, _L):
                t = t + hist[pl.ds(l * 256 + g * _L, _L)]
            rs = lax.rev(plsc.cumsum(lax.rev(t, (0,))), (0,))
            s = rs + above
            h = jnp.sum((s >= _K).astype(jnp.int32))
            hit = (found == 0) & (h > 0)
            bb = jnp.where(hit, g * _L + h - 1, bb)
            cnt_here = jnp.sum(jnp.where(lane >= h, t, 0))
            cntgt = jnp.where(hit, above + cnt_here, cntgt)
            found = jnp.where(hit, jnp.int32(1), found)
            above = above + jnp.sum(t)
            return found, bb, cntgt, above

        _, bb, cnt_gt, _ = lax.fori_loop(
            0, 16, pb,
            (jnp.int32(0), jnp.int32(0), jnp.int32(0), jnp.int32(0)))
        need = _K - cnt_gt

        def pe(v, off):
            k = krow[pl.ds(v * _L, _L)]
            b8 = (k >> 24) + 128
            orow[pl.ds(v * _L, _L)] = jnp.where(
                b8 > bb, jnp.maximum(k, 0), 0)
            meq = b8 == bb
            idx = off + plsc.cumsum(meq.astype(jnp.int32)) - 1
            plsc.store_scatter(ck, [idx], k, mask=meq)
            plsc.store_scatter(cc, [idx], v * _L + lane, mask=meq)
            return off + plsc.all_reduce_population_count(meq)

        ncand = lax.fori_loop(0, _NV, pe, zeros16, unroll=4)
        n_sc = jnp.max(ncand)
        nv_c = (n_sc + _L - 1) // _L

        def count_cmp(c, strict):
            def inner(i, acc):
                kv = ck[pl.ds(i * _L, _L)]
                valid = (i * _L + lane) < ncand
                m = ((kv > c) if strict else (kv >= c)) & valid
                return acc + m.astype(jnp.int32)

            return jnp.sum(lax.fori_loop(0, nv_c, inner, zeros16))

        def pd(j, p):
            c = p | (jnp.int32(1) << (23 - j))
            return jnp.where(count_cmp(c, False) >= need, c, p)

        tkey = lax.fori_loop(0, 24, pd, (bb - 128) << 24)
        n_eq = need - count_cmp(tkey, True)

        def pf(i, racc):
            kv = ck[pl.ds(i * _L, _L)]
            cv = cc[pl.ds(i * _L, _L)]
            valid = (i * _L + lane) < ncand
            meq = (kv == tkey) & valid
            inc_eq = meq & ((racc + plsc.cumsum(meq.astype(jnp.int32)))
                            <= n_eq)
            mwin = ((kv > tkey) & valid) | inc_eq
            plsc.store_scatter(orow, [cv], jnp.maximum(kv, 0), mask=mwin)
            return racc + plsc.all_reduce_population_count(meq)

        lax.fori_loop(0, nv_c, pf, zeros16)
        pltpu.sync_copy(orow, o_hbm.at[row])
        return _

    lax.fori_loop(0, 2, do_row, 0)


def _build(interpret=False):
    mesh = plsc.VectorSubcoreMesh(
        core_axis_name="c", subcore_axis_name="s",
        num_cores=2, num_subcores=16)
    return pl.kernel(
        _sc_body,
        out_type=jax.ShapeDtypeStruct((_R, _N), jnp.int32),
        mesh=mesh,
        scratch_types=[
            pltpu.VMEM((_N,), jnp.int32),
            pltpu.VMEM((_N,), jnp.int32),
            pltpu.VMEM((16 * 256,), jnp.int32),
            pltpu.VMEM((_N,), jnp.int32),
            pltpu.VMEM((_N,), jnp.int32),
        ],
        compiler_params=pltpu.CompilerParams(needs_layout_passes=False),
        interpret=interpret,
    )


@jax.jit
def kernel(x):
    xi = jax.lax.bitcast_convert_type(x, jnp.int32)
    oi = _build()(xi)
    return jax.lax.bitcast_convert_type(oi, jnp.float32)


# speedup vs baseline: 1.4554x; 1.4312x over previous
import jax
import jax.numpy as jnp
from jax import lax
from jax.experimental import pallas as pl
from jax.experimental.pallas import tpu as pltpu
from jax.experimental.pallas import tpu_sc as plsc

_K = 256
_R = 64
_N = 8192
_L = 16
_NV = _N // _L
_M31 = 0x7FFFFFFF


def _keys(b):
    return b ^ ((b >> 31) & _M31)


def _scan256(hist, rank):
    def ssb(i, carry):
        found, sbv, above_sel, above = carry
        sb = 15 - i
        t = hist[pl.ds(sb * 256, _L)]
        for j in range(1, _L):
            t = t + hist[pl.ds(sb * 256 + j * _L, _L)]
        s = jnp.sum(t)
        hit = (found == 0) & (above + s >= rank)
        sbv = jnp.where(hit, sb, sbv)
        above_sel = jnp.where(hit, above, above_sel)
        found = jnp.where(hit, jnp.int32(1), found)
        above = above + jnp.where(found == 0, s, 0)
        return found, sbv, above_sel, above

    z = jnp.int32(0)
    _, sbv, above_sel, _ = lax.fori_loop(0, 16, ssb, (z, z, z, z))

    def sj(i, carry):
        found, bv, cnt = carry
        b = sbv * _L + (15 - i)
        t = jnp.sum(hist[pl.ds(b * _L, _L)])
        hit = (found == 0) & (cnt + t >= rank)
        bv = jnp.where(hit, b, bv)
        found = jnp.where(hit, jnp.int32(1), found)
        cnt = cnt + jnp.where(found == 0, t, 0)
        return found, bv, cnt

    _, bv, cnt = lax.fori_loop(0, 16, sj, (z, z, above_sel))
    return bv, cnt


def _sc_body(x_hbm, o_hbm, krow, orow, hist, ck, cc, ck2, cc2):
    wid = lax.axis_index("s") * 2 + lax.axis_index("c")
    lane = lax.iota(jnp.int32, _L)
    ones16 = jnp.ones((_L,), jnp.int32)
    zeros16 = jnp.zeros((_L,), jnp.int32)

    def zero_hist():
        def zh(i, c):
            hist[pl.ds(i * _L, _L)] = zeros16
            return c

        lax.fori_loop(0, 256, zh, 0, unroll=8)

    def do_row(r, _):
        row = wid * 2 + r
        pltpu.sync_copy(x_hbm.at[row], krow)
        zero_hist()

        def pa(v, c):
            k = _keys(krow[pl.ds(v * _L, _L)])
            krow[pl.ds(v * _L, _L)] = k
            b8 = (k >> 24) + 128
            plsc.addupdate_scatter(hist, [b8 * _L + lane], ones16)
            return c

        lax.fori_loop(0, _NV, pa, 0, unroll=8)

        bb, cnt_gt = _scan256(hist, _K)
        need = _K - cnt_gt
        zero_hist()

        def pe(v, off):
            k = krow[pl.ds(v * _L, _L)]
            b8 = (k >> 24) + 128
            orow[pl.ds(v * _L, _L)] = jnp.where(
                b8 > bb, jnp.maximum(k, 0), 0)
            meq = b8 == bb
            idx = off + plsc.cumsum(meq.astype(jnp.int32)) - 1
            plsc.store_scatter(ck, [idx], k, mask=meq)
            plsc.store_scatter(cc, [idx], v * _L + lane, mask=meq)
            byte2 = (k >> 16) & 0xFF
            plsc.addupdate_scatter(hist, [byte2 * _L + lane], ones16,
                                   mask=meq)
            return off + plsc.all_reduce_population_count(meq)

        ncand = lax.fori_loop(0, _NV, pe, zeros16, unroll=4)
        nv_c = (jnp.max(ncand) + _L - 1) // _L

        b2, cnt_gt2 = _scan256(hist, need)
        need2 = need - cnt_gt2

        def pl2(i, off2):
            kv = ck[pl.ds(i * _L, _L)]
            cv = cc[pl.ds(i * _L, _L)]
            valid = (i * _L + lane) < ncand
            byte2 = (kv >> 16) & 0xFF
            plsc.store_scatter(orow, [cv], jnp.maximum(kv, 0),
                               mask=(byte2 > b2) & valid)
            m2 = (byte2 == b2) & valid
            idx = off2 + plsc.cumsum(m2.astype(jnp.int32)) - 1
            plsc.store_scatter(ck2, [idx], kv, mask=m2)
            plsc.store_scatter(cc2, [idx], cv, mask=m2)
            return off2 + plsc.all_reduce_population_count(m2)

        ncand2 = lax.fori_loop(0, nv_c, pl2, zeros16)
        nv2 = (jnp.max(ncand2) + _L - 1) // _L

        def count_cmp(c, strict):
            def inner(i, acc):
                kv = ck2[pl.ds(i * _L, _L)]
                valid = (i * _L + lane) < ncand2
                m = ((kv > c) if strict else (kv >= c)) & valid
                return acc + m.astype(jnp.int32)

            return jnp.sum(lax.fori_loop(0, nv2, inner, zeros16))

        def pd(j, p):
            c = p | (jnp.int32(1) << (15 - j))
            return jnp.where(count_cmp(c, False) >= need2, c, p)

        tkey = lax.fori_loop(0, 16, pd, ((bb - 128) << 24) | (b2 << 16))
        n_eq = need2 - count_cmp(tkey, True)

        def pf(i, racc):
            kv = ck2[pl.ds(i * _L, _L)]
            cv = cc2[pl.ds(i * _L, _L)]
            valid = (i * _L + lane) < ncand2
            meq = (kv == tkey) & valid
            inc_eq = meq & ((racc + plsc.cumsum(meq.astype(jnp.int32)))
                            <= n_eq)
            mwin = ((kv > tkey) & valid) | inc_eq
            plsc.store_scatter(orow, [cv], jnp.maximum(kv, 0), mask=mwin)
            return racc + plsc.all_reduce_population_count(meq)

        lax.fori_loop(0, nv2, pf, zeros16)
        pltpu.sync_copy(orow, o_hbm.at[row])
        return _

    lax.fori_loop(0, 2, do_row, 0)


def _build(interpret=False):
    mesh = plsc.VectorSubcoreMesh(
        core_axis_name="c", subcore_axis_name="s",
        num_cores=2, num_subcores=16)
    return pl.kernel(
        _sc_body,
        out_type=jax.ShapeDtypeStruct((_R, _N), jnp.int32),
        mesh=mesh,
        scratch_types=[
            pltpu.VMEM((_N,), jnp.int32),
            pltpu.VMEM((_N,), jnp.int32),
            pltpu.VMEM((16 * 256,), jnp.int32),
            pltpu.VMEM((_N,), jnp.int32),
            pltpu.VMEM((_N,), jnp.int32),
            pltpu.VMEM((_N,), jnp.int32),
            pltpu.VMEM((_N,), jnp.int32),
        ],
        compiler_params=pltpu.CompilerParams(needs_layout_passes=False),
        interpret=interpret,
    )


@jax.jit
def kernel(x):
    xi = jax.lax.bitcast_convert_type(x, jnp.int32)
    oi = _build()(xi)
    return jax.lax.bitcast_convert_type(oi, jnp.float32)


# speedup vs baseline: 1.9306x; 1.3265x over previous
import jax
import jax.numpy as jnp
from jax import lax
from jax.experimental import pallas as pl
from jax.experimental.pallas import tpu as pltpu
from jax.experimental.pallas import tpu_sc as plsc

_K = 256
_R = 64
_N = 8192
_L = 16
_NV = _N // _L
_M31 = 0x7FFFFFFF


def _keys(b):
    return b ^ ((b >> 31) & _M31)


def _scan256(hist, rank):
    def ssb(i, carry):
        found, sbv, above_sel, above = carry
        sb = 15 - i
        t = hist[pl.ds(sb * 256, _L)]
        for j in range(1, _L):
            t = t + hist[pl.ds(sb * 256 + j * _L, _L)]
        s = jnp.sum(t)
        hit = (found == 0) & (above + s >= rank)
        sbv = jnp.where(hit, sb, sbv)
        above_sel = jnp.where(hit, above, above_sel)
        found = jnp.where(hit, jnp.int32(1), found)
        above = above + jnp.where(found == 0, s, 0)
        return found, sbv, above_sel, above

    z = jnp.int32(0)
    _, sbv, above_sel, _ = lax.fori_loop(0, 16, ssb, (z, z, z, z))

    def sj(i, carry):
        found, bv, cnt = carry
        b = sbv * _L + (15 - i)
        t = jnp.sum(hist[pl.ds(b * _L, _L)])
        hit = (found == 0) & (cnt + t >= rank)
        bv = jnp.where(hit, b, bv)
        found = jnp.where(hit, jnp.int32(1), found)
        cnt = cnt + jnp.where(found == 0, t, 0)
        return found, bv, cnt

    _, bv, cnt = lax.fori_loop(0, 16, sj, (z, z, above_sel))
    return bv, cnt


def _sc_body(x_hbm, o_hbm, krow0, krow1, orow0, orow1, hist, cc, ck2, cc2,
             sem0, sem1, sem2):
    wid = lax.axis_index("s") * 2 + lax.axis_index("c")
    lane = lax.iota(jnp.int32, _L)
    ones16 = jnp.ones((_L,), jnp.int32)
    zeros16 = jnp.zeros((_L,), jnp.int32)

    def zero_hist():
        def zh(i, c):
            hist[pl.ds(i * _L, _L)] = zeros16
            return c

        lax.fori_loop(0, 256, zh, 0, unroll=8)

    def process(krow, orow):
        zero_hist()

        @plsc.parallel_loop(0, _NV, unroll=8)
        def pa(v):
            k = _keys(krow[pl.ds(v * _L, _L)])
            krow[pl.ds(v * _L, _L)] = k
            b8 = (k >> 24) + 128
            plsc.addupdate_scatter(hist, [b8 * _L + lane], ones16)

        bb, cnt_gt = _scan256(hist, _K)
        need = _K - cnt_gt
        zero_hist()

        @plsc.parallel_loop(0, _NV, unroll=4, carry=zeros16)
        def pe(v, off):
            k = krow[pl.ds(v * _L, _L)]
            b8 = (k >> 24) + 128
            orow[pl.ds(v * _L, _L)] = jnp.where(
                b8 > bb, jnp.maximum(k, 0), 0)
            meq = b8 == bb
            idx = off + plsc.cumsum(meq.astype(jnp.int32)) - 1
            plsc.store_scatter(cc, [idx], v * _L + lane, mask=meq)
            return off + plsc.all_reduce_population_count(meq)

        ncand = pe
        nv_c = (jnp.max(ncand) + _L - 1) // _L

        def p25(i, c):
            cv = cc[pl.ds(i * _L, _L)]
            valid = (i * _L + lane) < ncand
            kv = plsc.load_gather(krow, [cv], mask=valid)
            byte2 = (kv >> 16) & 0xFF
            plsc.addupdate_scatter(hist, [byte2 * _L + lane], ones16,
                                   mask=valid)
            return c

        lax.fori_loop(0, nv_c, p25, 0)

        b2, cnt_gt2 = _scan256(hist, need)
        need2 = need - cnt_gt2

        def pl2(i, off2):
            cv = cc[pl.ds(i * _L, _L)]
            valid = (i * _L + lane) < ncand
            kv = plsc.load_gather(krow, [cv], mask=valid)
            byte2 = (kv >> 16) & 0xFF
            plsc.store_scatter(orow, [cv], jnp.maximum(kv, 0),
                               mask=(byte2 > b2) & valid)
            m2 = (byte2 == b2) & valid
            idx = off2 + plsc.cumsum(m2.astype(jnp.int32)) - 1
            plsc.store_scatter(ck2, [idx], kv, mask=m2)
            plsc.store_scatter(cc2, [idx], cv, mask=m2)
            return off2 + plsc.all_reduce_population_count(m2)

        ncand2 = lax.fori_loop(0, nv_c, pl2, zeros16)
        nv2 = (jnp.max(ncand2) + _L - 1) // _L

        def count_cmp(c, strict):
            def inner(i, acc):
                kv = ck2[pl.ds(i * _L, _L)]
                valid = (i * _L + lane) < ncand2
                m = ((kv > c) if strict else (kv >= c)) & valid
                return acc + m.astype(jnp.int32)

            return jnp.sum(lax.fori_loop(0, nv2, inner, zeros16))

        def pd(j, p):
            c = p | (jnp.int32(1) << (15 - j))
            return jnp.where(count_cmp(c, False) >= need2, c, p)

        tkey = lax.fori_loop(0, 16, pd, ((bb - 128) << 24) | (b2 << 16))
        n_eq = need2 - count_cmp(tkey, True)

        def pf(i, racc):
            kv = ck2[pl.ds(i * _L, _L)]
            cv = cc2[pl.ds(i * _L, _L)]
            valid = (i * _L + lane) < ncand2
            meq = (kv == tkey) & valid
            inc_eq = meq & ((racc + plsc.cumsum(meq.astype(jnp.int32)))
                            <= n_eq)
            mwin = ((kv > tkey) & valid) | inc_eq
            plsc.store_scatter(orow, [cv], jnp.maximum(kv, 0), mask=mwin)
            return racc + plsc.all_reduce_population_count(meq)

        lax.fori_loop(0, nv2, pf, zeros16)

    row0 = wid * 2
    row1 = row0 + 1
    cp0 = pltpu.async_copy(x_hbm.at[row0], krow0, sem0)
    cp1 = pltpu.async_copy(x_hbm.at[row1], krow1, sem1)
    cp0.wait()
    process(krow0, orow0)
    co0 = pltpu.async_copy(orow0, o_hbm.at[row0], sem2)
    cp1.wait()
    process(krow1, orow1)
    co0.wait()
    pltpu.sync_copy(orow1, o_hbm.at[row1])


def _build(interpret=False):
    mesh = plsc.VectorSubcoreMesh(
        core_axis_name="c", subcore_axis_name="s",
        num_cores=2, num_subcores=16)
    return pl.kernel(
        _sc_body,
        out_type=jax.ShapeDtypeStruct((_R, _N), jnp.int32),
        mesh=mesh,
        scratch_types=[
            pltpu.VMEM((_N,), jnp.int32),
            pltpu.VMEM((_N,), jnp.int32),
            pltpu.VMEM((_N,), jnp.int32),
            pltpu.VMEM((_N,), jnp.int32),
            pltpu.VMEM((16 * 256,), jnp.int32),
            pltpu.VMEM((_N,), jnp.int32),
            pltpu.VMEM((_N,), jnp.int32),
            pltpu.VMEM((_N,), jnp.int32),
            pltpu.SemaphoreType.DMA,
            pltpu.SemaphoreType.DMA,
            pltpu.SemaphoreType.DMA,
        ],
        compiler_params=pltpu.CompilerParams(needs_layout_passes=False),
        interpret=interpret,
    )


@jax.jit
def kernel(x):
    xi = jax.lax.bitcast_convert_type(x, jnp.int32)
    oi = _build()(xi)
    return jax.lax.bitcast_convert_type(oi, jnp.float32)


# speedup vs baseline: 2.5454x; 1.3185x over previous
import jax
import jax.numpy as jnp
from jax import lax
from jax.experimental import pallas as pl
from jax.experimental.pallas import tpu as pltpu
from jax.experimental.pallas import tpu_sc as plsc

_K = 256
_R = 64
_N = 8192
_L = 16
_NV = _N // _L
_M31 = 0x7FFFFFFF


def _keys(b):
    return b ^ ((b >> 31) & _M31)


def _scan256(hist, rank):
    def ssb(i, carry):
        found, sbv, above_sel, above = carry
        sb = 15 - i
        t = hist[pl.ds(sb * 256, _L)]
        for j in range(1, _L):
            t = t + hist[pl.ds(sb * 256 + j * _L, _L)]
        s = jnp.sum(t)
        hit = (found == 0) & (above + s >= rank)
        sbv = jnp.where(hit, sb, sbv)
        above_sel = jnp.where(hit, above, above_sel)
        found = jnp.where(hit, jnp.int32(1), found)
        above = above + jnp.where(found == 0, s, 0)
        return found, sbv, above_sel, above

    z = jnp.int32(0)
    _, sbv, above_sel, _ = lax.fori_loop(0, 16, ssb, (z, z, z, z))

    def sj(i, carry):
        found, bv, cnt = carry
        b = sbv * _L + (15 - i)
        t = jnp.sum(hist[pl.ds(b * _L, _L)])
        hit = (found == 0) & (cnt + t >= rank)
        bv = jnp.where(hit, b, bv)
        found = jnp.where(hit, jnp.int32(1), found)
        cnt = cnt + jnp.where(found == 0, t, 0)
        return found, bv, cnt

    _, bv, cnt = lax.fori_loop(0, 16, sj, (z, z, above_sel))
    return bv, cnt


def _sc_body(x_hbm, o_hbm, krow0, krow1, orow0, orow1, hist, cc, ck2, cc2,
             sem0, sem1, sem2):
    wid = lax.axis_index("s") * 2 + lax.axis_index("c")
    lane = lax.iota(jnp.int32, _L)
    ones16 = jnp.ones((_L,), jnp.int32)
    zeros16 = jnp.zeros((_L,), jnp.int32)

    def zero_hist():
        def zh(i, c):
            hist[pl.ds(i * _L, _L)] = zeros16
            return c

        lax.fori_loop(0, 256, zh, 0, unroll=8)

    def process(krow, orow):
        zero_hist()

        @plsc.parallel_loop(0, _NV, unroll=8)
        def pa(v):
            k = _keys(plsc.bitcast(krow[pl.ds(v * _L, _L)], jnp.int32))
            krow[pl.ds(v * _L, _L)] = plsc.bitcast(k, jnp.float32)
            b8 = (k >> 24) + 128
            plsc.addupdate_scatter(hist, [b8 * _L + lane], ones16)

        bb, cnt_gt = _scan256(hist, _K)
        need = _K - cnt_gt
        zero_hist()

        @plsc.parallel_loop(0, _NV, unroll=4, carry=zeros16)
        def pe(v, off):
            k = plsc.bitcast(krow[pl.ds(v * _L, _L)], jnp.int32)
            b8 = (k >> 24) + 128
            orow[pl.ds(v * _L, _L)] = plsc.bitcast(
                jnp.where(b8 > bb, jnp.maximum(k, 0), 0), jnp.float32)
            meq = b8 == bb
            idx = off + plsc.cumsum(meq.astype(jnp.int32)) - 1
            plsc.store_scatter(cc, [idx], v * _L + lane, mask=meq)
            return off + plsc.all_reduce_population_count(meq)

        ncand = pe
        nv_c = (jnp.max(ncand) + _L - 1) // _L

        @plsc.parallel_loop(0, nv_c, unroll=2)
        def p25(i):
            cv = cc[pl.ds(i * _L, _L)]
            valid = (i * _L + lane) < ncand
            kv = plsc.bitcast(plsc.load_gather(krow, [cv], mask=valid),
                              jnp.int32)
            byte2 = (kv >> 16) & 0xFF
            plsc.addupdate_scatter(hist, [byte2 * _L + lane], ones16,
                                   mask=valid)

        b2, cnt_gt2 = _scan256(hist, need)
        need2 = need - cnt_gt2

        @plsc.parallel_loop(0, nv_c, unroll=2, carry=zeros16)
        def pl2(i, off2):
            cv = cc[pl.ds(i * _L, _L)]
            valid = (i * _L + lane) < ncand
            kv = plsc.bitcast(plsc.load_gather(krow, [cv], mask=valid),
                              jnp.int32)
            byte2 = (kv >> 16) & 0xFF
            plsc.store_scatter(orow, [cv],
                               plsc.bitcast(jnp.maximum(kv, 0),
                                            jnp.float32),
                               mask=(byte2 > b2) & valid)
            m2 = (byte2 == b2) & valid
            idx = off2 + plsc.cumsum(m2.astype(jnp.int32)) - 1
            plsc.store_scatter(ck2, [idx], kv, mask=m2)
            plsc.store_scatter(cc2, [idx], cv, mask=m2)
            return off2 + plsc.all_reduce_population_count(m2)

        ncand2 = pl2
        nv2 = (jnp.max(ncand2) + _L - 1) // _L

        def count_cmp(c, strict):
            def inner(i, acc):
                kv = ck2[pl.ds(i * _L, _L)]
                valid = (i * _L + lane) < ncand2
                m = ((kv > c) if strict else (kv >= c)) & valid
                return acc + m.astype(jnp.int32)

            return jnp.sum(lax.fori_loop(0, nv2, inner, zeros16))

        def pd(j, p):
            c = p | (jnp.int32(1) << (15 - j))
            return jnp.where(count_cmp(c, False) >= need2, c, p)

        tkey = lax.fori_loop(0, 16, pd, ((bb - 128) << 24) | (b2 << 16))
        n_eq = need2 - count_cmp(tkey, True)

        def pf(i, racc):
            kv = ck2[pl.ds(i * _L, _L)]
            cv = cc2[pl.ds(i * _L, _L)]
            valid = (i * _L + lane) < ncand2
            meq = (kv == tkey) & valid
            inc_eq = meq & ((racc + plsc.cumsum(meq.astype(jnp.int32)))
                            <= n_eq)
            mwin = ((kv > tkey) & valid) | inc_eq
            plsc.store_scatter(orow, [cv],
                               plsc.bitcast(jnp.maximum(kv, 0),
                                            jnp.float32), mask=mwin)
            return racc + plsc.all_reduce_population_count(meq)

        lax.fori_loop(0, nv2, pf, zeros16)

    row0 = wid * 2
    row1 = row0 + 1
    cp0 = pltpu.async_copy(x_hbm.at[row0], krow0, sem0)
    cp1 = pltpu.async_copy(x_hbm.at[row1], krow1, sem1)
    cp0.wait()
    process(krow0, orow0)
    co0 = pltpu.async_copy(orow0, o_hbm.at[row0], sem2)
    cp1.wait()
    process(krow1, orow1)
    co0.wait()
    pltpu.sync_copy(orow1, o_hbm.at[row1])


def _build(interpret=False):
    mesh = plsc.VectorSubcoreMesh(
        core_axis_name="c", subcore_axis_name="s",
        num_cores=2, num_subcores=16)
    return pl.kernel(
        _sc_body,
        out_type=jax.ShapeDtypeStruct((_R, _N), jnp.float32),
        mesh=mesh,
        scratch_types=[
            pltpu.VMEM((_N,), jnp.float32),
            pltpu.VMEM((_N,), jnp.float32),
            pltpu.VMEM((_N,), jnp.float32),
            pltpu.VMEM((_N,), jnp.float32),
            pltpu.VMEM((16 * 256,), jnp.int32),
            pltpu.VMEM((_N,), jnp.int32),
            pltpu.VMEM((_N,), jnp.int32),
            pltpu.VMEM((_N,), jnp.int32),
            pltpu.SemaphoreType.DMA,
            pltpu.SemaphoreType.DMA,
            pltpu.SemaphoreType.DMA,
        ],
        compiler_params=pltpu.CompilerParams(needs_layout_passes=False),
        interpret=interpret,
    )


@jax.jit
def kernel(x):
    return _build()(x)
